# baseline (speedup 1.0000x reference)
import dataclasses
import functools

import jax
import jax.numpy as jnp
from jax import lax
from jax.experimental import pallas as pl
from jax.experimental.pallas import tpu as pltpu
from jax.experimental.pallas import tpu_sc as plsc

_N = 10000
_E = 320000
_D = 128
_NCTR = 512

_NCORES = 2
_NSUB = 16
_NW = _NCORES * _NSUB
_EPT = _E // _NW
_K = 128
_NFULL = _EPT // _K
_KTAIL = _EPT - _NFULL * _K
_RPT = 624
_RREM = _N - _NSUB * _RPT


def _sc_mesh():
    return plsc.VectorSubcoreMesh(
        core_axis_name="c", subcore_axis_name="s", num_cores=_NCORES,
        num_subcores=_NSUB)


def _sc_params(layout_passes=True):
    cp = pltpu.CompilerParams(disable_bounds_checks=True)
    if not layout_passes and (
            "needs_layout_passes" in pltpu.CompilerParams.__dataclass_fields__):
        cp = dataclasses.replace(cp, needs_layout_passes=False)
    return cp


def _seg_sum(h, src, dst, zeros):
    nrow = 2
    nidx = 4

    @functools.partial(
        pl.kernel,
        out_type=jax.ShapeDtypeStruct((_NCORES, _N, _D), jnp.float32),
        mesh=_sc_mesh(),
        compiler_params=_sc_params(),
        scratch_types=[
            [pltpu.VMEM((_K,), jnp.int32) for _ in range(nidx)],
            [pltpu.VMEM((_K,), jnp.int32) for _ in range(nidx)],
            [pltpu.VMEM((_K, _D), jnp.float32) for _ in range(nrow)],
            [pltpu.SemaphoreType.DMA for _ in range(nidx)],
            [pltpu.SemaphoreType.DMA for _ in range(nrow)],
            pltpu.VMEM((_KTAIL,), jnp.int32),
            pltpu.VMEM((_KTAIL,), jnp.int32),
            pltpu.VMEM((_KTAIL, _D), jnp.float32),
            pltpu.VMEM_SHARED((_N, _D), jnp.float32),
        ],
    )
    def k(h_hbm, src_hbm, dst_hbm, z_hbm, out_hbm, src_v, dst_v, rows_v,
          isem, gsem, srct_v, dstt_v, rowst_v, acc):
        cid = lax.axis_index("c")
        sid = lax.axis_index("s")
        wid = cid * _NSUB + sid
        r0 = pl.multiple_of(sid * _RPT, 8)
        pltpu.sync_copy(z_hbm.at[pl.ds(r0, _RPT)], acc.at[pl.ds(r0, _RPT)])

        @pl.when(sid == 0)
        def _():
            pltpu.sync_copy(z_hbm.at[pl.ds(_NSUB * _RPT, _RREM)],
                            acc.at[pl.ds(_NSUB * _RPT, _RREM)])

        plsc.subcore_barrier()
        base = wid * _EPT

        def issue_idx(c, bi):
            off = pl.multiple_of(base + c * _K, 8)
            pltpu.async_copy(src_hbm.at[pl.ds(off, _K)], src_v[bi], isem[bi])
            pltpu.async_copy(dst_hbm.at[pl.ds(off, _K)], dst_v[bi], isem[bi])

        def issue_gather(bi, br):
            pltpu.make_async_copy(src_hbm.at[pl.ds(0, _K)], src_v[bi],
                                  isem[bi]).wait()
            pltpu.make_async_copy(dst_hbm.at[pl.ds(0, _K)], dst_v[bi],
                                  isem[bi]).wait()
            pltpu.async_copy(h_hbm.at[src_v[bi]], rows_v[br], gsem[br])

        def drain(bi, br):
            pltpu.make_async_copy(h_hbm.at[src_v[bi]], rows_v[br],
                                  gsem[br]).wait()
            pltpu.sync_copy(rows_v[br], acc.at[dst_v[bi]], add=True)

        for c in range(nidx):
            issue_idx(c, c)
        issue_gather(0, 0)

        @pl.loop(0, 17)
        def _(j):
            c0 = 4 * j
            for b in range(4):
                issue_gather((b + 1) % nidx, (b + 1) % nrow)
                drain(b, b % nrow)
                issue_idx(c0 + b + 4, b)

        for c in range(68, _NFULL):
            if c + 1 < _NFULL:
                issue_gather((c + 1) % nidx, (c + 1) % nrow)
            drain(c % nidx, c % nrow)
            if c + 4 < _NFULL:
                issue_idx(c + 4, c % nidx)

        offt = pl.multiple_of(base + _NFULL * _K, 8)
        pltpu.sync_copy(src_hbm.at[pl.ds(offt, _KTAIL)], srct_v)
        pltpu.sync_copy(dst_hbm.at[pl.ds(offt, _KTAIL)], dstt_v)
        pltpu.sync_copy(h_hbm.at[srct_v], rowst_v)
        pltpu.sync_copy(rowst_v, acc.at[dstt_v], add=True)

        plsc.subcore_barrier()
        pltpu.sync_copy(acc.at[pl.ds(r0, _RPT)], out_hbm.at[cid, pl.ds(r0, _RPT)])

        @pl.when(sid == 0)
        def _():
            pltpu.sync_copy(acc.at[pl.ds(_NSUB * _RPT, _RREM)],
                            out_hbm.at[cid, pl.ds(_NSUB * _RPT, _RREM)])

    return k(h, src, dst, zeros)


_SCN = 1024
_SCNF = _EPT // _SCN
_SCNT = _EPT - _SCNF * _SCN
_MCAP = 10496
_CROWS = 520
_TRASH = _NCTR * 1024 + _NCTR


def _center_stage(h2, zx, src, dst, centers, zeros, zeros_i, trash_i):
    bpw = _NCTR // _NW
    bps = _NCTR // _NSUB

    @functools.partial(
        pl.kernel,
        compiler_params=_sc_params(layout_passes=False),
        out_type=[
            jax.ShapeDtypeStruct((_NCORES * _NCTR, _D), jnp.float32),
            jax.ShapeDtypeStruct((_NCORES, _NCTR, _D), jnp.float32),
            jax.ShapeDtypeStruct((_NCTR, _D), jnp.float32),
            jax.ShapeDtypeStruct((_NCTR, _D), jnp.float32),
        ],
        mesh=_sc_mesh(),
        scratch_types=[
            pltpu.VMEM((_N,), jnp.int32),
            pltpu.VMEM((_NCTR,), jnp.int32),
            [pltpu.VMEM((_SCN,), jnp.int32) for _ in range(2)],
            [pltpu.VMEM((_SCN,), jnp.int32) for _ in range(2)],
            [pltpu.SemaphoreType.DMA for _ in range(2)],
            pltpu.VMEM((_MCAP,), jnp.int32),
            [pltpu.VMEM((_K,), jnp.int32) for _ in range(2)],
            [pltpu.VMEM((_K,), jnp.int32) for _ in range(2)],
            [pltpu.VMEM((_K, _D), jnp.float32) for _ in range(2)],
            [pltpu.SemaphoreType.DMA for _ in range(2)],
            pltpu.VMEM((bps,), jnp.int32),
            pltpu.VMEM((bps, _D), jnp.float32),
            pltpu.VMEM((bps, _D), jnp.float32),
            pltpu.VMEM_SHARED((_CROWS, _D), jnp.float32),
        ],
    )
    def k(h2_hbm, zx_hbm, src_hbm, dst_hbm, ctr_hbm, z_hbm, zi_hbm, tr_hbm,
          raw_hbm, rem_hbm, zxc_hbm, h2c_hbm,
          marker, ctr_v, ssrc_v, sdst_v, ssem, mpack_v, gsrc_v,
          gdst_v, rows_v, gsem, pmap_v, prow_v, crow_v, cagg):
        cid = lax.axis_index("c")
        sid = lax.axis_index("s")
        wid = cid * _NSUB + sid

        pltpu.sync_copy(zi_hbm, marker)
        pltpu.sync_copy(tr_hbm, mpack_v)
        pltpu.sync_copy(ctr_hbm, ctr_v)
        for j in range(_NCTR // 16):
            cvec = ctr_v[pl.ds(16 * j, 16)]
            vals = lax.iota(jnp.int32, 16) + (16 * j + 1)
            plsc.store_scatter(marker, [cvec], vals)
        r0 = pl.multiple_of(sid * bps, 8)
        pltpu.sync_copy(z_hbm.at[pl.ds(r0, bps)], cagg.at[pl.ds(r0, bps)])

        @pl.when(sid == 0)
        def _():
            pltpu.sync_copy(z_hbm.at[pl.ds(_NCTR, _CROWS - _NCTR)],
                            cagg.at[pl.ds(_NCTR, _CROWS - _NCTR)])

        plsc.subcore_barrier()

        base = wid * _EPT

        def make_scan_sub(b):
            def scan_sub(u, cnt):
                dvec = sdst_v[b][pl.ds(16 * u, 16)]
                svec = ssrc_v[b][pl.ds(16 * u, 16)]
                m = plsc.load_gather(marker, [dvec])
                mask = m > 0
                packed = svec * 1024 + (m - 1)
                plsc.store_compressed(mpack_v.at[pl.ds(cnt, 16)], packed,
                                      mask=mask)
                return cnt + jnp.sum(mask.astype(jnp.int32))
            return scan_sub

        def issue_scan(ci, b):
            off = pl.multiple_of(base + ci * _SCN, 8)
            pltpu.async_copy(src_hbm.at[pl.ds(off, _SCN)], ssrc_v[b], ssem[b])
            pltpu.async_copy(dst_hbm.at[pl.ds(off, _SCN)], sdst_v[b], ssem[b])

        def wait_scan(b):
            pltpu.make_async_copy(src_hbm.at[pl.ds(0, _SCN)], ssrc_v[b],
                                  ssem[b]).wait()
            pltpu.make_async_copy(dst_hbm.at[pl.ds(0, _SCN)], sdst_v[b],
                                  ssem[b]).wait()

        def issue_scan_tail(b):
            offt = pl.multiple_of(base + _SCNF * _SCN, 8)
            pltpu.async_copy(src_hbm.at[pl.ds(offt, _SCNT)],
                             ssrc_v[b].at[pl.ds(0, _SCNT)], ssem[b])
            pltpu.async_copy(dst_hbm.at[pl.ds(offt, _SCNT)],
                             sdst_v[b].at[pl.ds(0, _SCNT)], ssem[b])

        def wait_scan_tail(b):
            pltpu.make_async_copy(src_hbm.at[pl.ds(0, _SCNT)],
                                  ssrc_v[b].at[pl.ds(0, _SCNT)],
                                  ssem[b]).wait()
            pltpu.make_async_copy(dst_hbm.at[pl.ds(0, _SCNT)],
                                  sdst_v[b].at[pl.ds(0, _SCNT)],
                                  ssem[b]).wait()

        issue_scan(0, 0)
        issue_scan(1, 1)
        cnt = jnp.int32(0)
        for ci in range(_SCNF):
            b = ci % 2
            wait_scan(b)
            cnt = lax.fori_loop(0, _SCN // 16, make_scan_sub(b), cnt)
            if ci + 2 < _SCNF:
                issue_scan(ci + 2, b)
            elif ci + 2 == _SCNF:
                issue_scan_tail(b)
        bt = _SCNF % 2
        wait_scan_tail(bt)
        cnt = lax.fori_loop(0, _SCNT // 16, make_scan_sub(bt), cnt)

        npair = (cnt + 2 * _K - 1) // (2 * _K)

        def decode_issue(q, b):
            qo = q * _K
            for v in range(_K // 16):
                pk = mpack_v[pl.ds(qo + 16 * v, 16)]
                gsrc_v[b][pl.ds(16 * v, 16)] = pk >> 10
                gdst_v[b][pl.ds(16 * v, 16)] = pk & 1023
            pltpu.async_copy(h2_hbm.at[gsrc_v[b]], rows_v[b], gsem[b])

        def drain_scatter(b):
            pltpu.make_async_copy(h2_hbm.at[gsrc_v[b]], rows_v[b],
                                  gsem[b]).wait()
            pltpu.sync_copy(rows_v[b], cagg.at[gdst_v[b]], add=True)

        decode_issue(0, 0)
        decode_issue(1, 1)

        @pl.loop(0, npair)
        def _(t):
            q = 2 * t
            drain_scatter(0)
            decode_issue(q + 2, 0)
            drain_scatter(1)
            decode_issue(q + 3, 1)

        pltpu.make_async_copy(h2_hbm.at[gsrc_v[0]], rows_v[0], gsem[0]).wait()
        pltpu.make_async_copy(h2_hbm.at[gsrc_v[1]], rows_v[1], gsem[1]).wait()

        plsc.subcore_barrier()

        pltpu.sync_copy(cagg.at[pl.ds(r0, bps)],
                        raw_hbm.at[pl.ds(cid * _NCTR + r0, bps)])
        plsc.subcore_barrier()

        pbase = pl.multiple_of(sid * bps, 8)
        for t in range(bps // 16):
            pvec = plsc.load_gather(
                marker, [ctr_v[pl.ds(pbase + 16 * t, 16)]]) - 1
            pmap_v[pl.ds(16 * t, 16)] = pvec + cid * _NCTR
        pltpu.sync_copy(raw_hbm.at[pmap_v], prow_v)
        pltpu.sync_copy(prow_v, rem_hbm.at[cid, pl.ds(pbase, bps)])

        @pl.when(cid == 0)
        def _():
            cbase = pl.multiple_of(sid * bps, 8)
            pltpu.sync_copy(zx_hbm.at[ctr_v.at[pl.ds(cbase, bps)]], crow_v)
            pltpu.sync_copy(crow_v, zxc_hbm.at[pl.ds(cbase, bps)])
            pltpu.sync_copy(h2_hbm.at[ctr_v.at[pl.ds(cbase, bps)]], crow_v)
            pltpu.sync_copy(crow_v, h2c_hbm.at[pl.ds(cbase, bps)])

    return k(h2, zx, src, dst, centers, zeros, zeros_i, trash_i)


_BR = 1000


def _emb_mm(x, W, b):
    def body(x_ref, w_ref, b_ref, o_ref):
        o_ref[...] = (
            jnp.dot(x_ref[...], w_ref[...], preferred_element_type=jnp.float32)
            + b_ref[...]
        )

    return pl.pallas_call(
        body,
        grid=(_N // _BR,),
        in_specs=[
            pl.BlockSpec((_BR, _D), lambda i: (i, 0)),
            pl.BlockSpec((_D, _D), lambda i: (0, 0)),
            pl.BlockSpec((1, _D), lambda i: (0, 0)),
        ],
        out_specs=pl.BlockSpec((_BR, _D), lambda i: (i, 0)),
        out_shape=jax.ShapeDtypeStruct((_N, _D), jnp.float32),
    )(x, W, b.reshape(1, _D))


def _gin0_mm(x_in, agg, W, b):
    def body(x_ref, a0_ref, a1_ref, w_ref, b_ref, z_ref, h_ref):
        z = x_ref[...] + a0_ref[0] + a1_ref[0]
        z_ref[...] = z
        h_ref[...] = jnp.maximum(
            jnp.dot(z, w_ref[...], preferred_element_type=jnp.float32)
            + b_ref[...],
            0.0,
        )

    return pl.pallas_call(
        body,
        grid=(_N // _BR,),
        in_specs=[
            pl.BlockSpec((_BR, _D), lambda i: (i, 0)),
            pl.BlockSpec((1, _BR, _D), lambda i: (0, i, 0)),
            pl.BlockSpec((1, _BR, _D), lambda i: (1, i, 0)),
            pl.BlockSpec((_D, _D), lambda i: (0, 0)),
            pl.BlockSpec((1, _D), lambda i: (0, 0)),
        ],
        out_specs=[
            pl.BlockSpec((_BR, _D), lambda i: (i, 0)),
            pl.BlockSpec((_BR, _D), lambda i: (i, 0)),
        ],
        out_shape=[
            jax.ShapeDtypeStruct((_N, _D), jnp.float32),
            jax.ShapeDtypeStruct((_N, _D), jnp.float32),
        ],
    )(x_in, agg, agg, W, b.reshape(1, _D))


def _gin_mm(zx, h, agg, Wa, Wb, b):
    def body(zx_ref, h_ref, a0_ref, a1_ref, wa_ref, wb_ref, b_ref, o_ref):
        zh = h_ref[...] + a0_ref[0] + a1_ref[0]
        acc = jnp.dot(zx_ref[...], wa_ref[...], preferred_element_type=jnp.float32)
        acc += jnp.dot(zh, wb_ref[...], preferred_element_type=jnp.float32)
        o_ref[...] = jnp.maximum(acc + b_ref[...], 0.0)

    return pl.pallas_call(
        body,
        grid=(_N // _BR,),
        in_specs=[
            pl.BlockSpec((_BR, _D), lambda i: (i, 0)),
            pl.BlockSpec((_BR, _D), lambda i: (i, 0)),
            pl.BlockSpec((1, _BR, _D), lambda i: (0, i, 0)),
            pl.BlockSpec((1, _BR, _D), lambda i: (1, i, 0)),
            pl.BlockSpec((_D, _D), lambda i: (0, 0)),
            pl.BlockSpec((_D, _D), lambda i: (0, 0)),
            pl.BlockSpec((1, _D), lambda i: (0, 0)),
        ],
        out_specs=pl.BlockSpec((_BR, _D), lambda i: (i, 0)),
        out_shape=jax.ShapeDtypeStruct((_N, _D), jnp.float32),
    )(zx, h, agg, agg, Wa, Wb, b.reshape(1, _D))


def _final_mm(zxc, h2c, caggc, Wa, Wb, bg, W0, b0, W1, b1):
    def body(zx_ref, h_ref, a0_ref, a1_ref, wa_ref, wb_ref, bg_ref,
             w0_ref, b0_ref, w1_ref, b1_ref, o_ref):
        zh = h_ref[...] + a0_ref[0] + a1_ref[0]
        acc = jnp.dot(zx_ref[...], wa_ref[...], preferred_element_type=jnp.float32)
        acc += jnp.dot(zh, wb_ref[...], preferred_element_type=jnp.float32)
        h3 = jnp.maximum(acc + bg_ref[...], 0.0)
        t = (
            jnp.dot(h3, w0_ref[...], preferred_element_type=jnp.float32)
            + b0_ref[...]
        )
        o_ref[...] = (
            jnp.dot(t, w1_ref[...], preferred_element_type=jnp.float32)
            + b1_ref[...]
        )

    return pl.pallas_call(
        body,
        grid=(1,),
        in_specs=[
            pl.BlockSpec((_NCTR, _D), lambda i: (0, 0)),
            pl.BlockSpec((_NCTR, _D), lambda i: (0, 0)),
            pl.BlockSpec((1, _NCTR, _D), lambda i: (0, 0, 0)),
            pl.BlockSpec((1, _NCTR, _D), lambda i: (1, 0, 0)),
            pl.BlockSpec((_D, _D), lambda i: (0, 0)),
            pl.BlockSpec((_D, _D), lambda i: (0, 0)),
            pl.BlockSpec((1, _D), lambda i: (0, 0)),
            pl.BlockSpec((_D, _D), lambda i: (0, 0)),
            pl.BlockSpec((1, _D), lambda i: (0, 0)),
            pl.BlockSpec((_D, 1), lambda i: (0, 0)),
            pl.BlockSpec((1, 1), lambda i: (0, 0)),
        ],
        out_specs=pl.BlockSpec((_NCTR, 1), lambda i: (0, 0)),
        out_shape=jax.ShapeDtypeStruct((_NCTR, 1), jnp.float32),
    )(zxc, h2c, caggc, caggc, Wa, Wb, bg.reshape(1, _D),
      W0, b0.reshape(1, _D), W1, b1.reshape(1, 1))


def kernel(x, edge_index, center_node_index, W_emb, b_emb, W_g0, b_g0,
           W_g1, b_g1, W_g2, b_g2, W_o0, b_o0, W_o1, b_o1):
    src = edge_index[0]
    dst = edge_index[1]
    zeros = jnp.zeros((_N, _D), jnp.float32)
    zeros_i = jnp.zeros((_N,), jnp.int32)
    trash_i = jnp.full((_MCAP,), _TRASH, jnp.int32)

    x_in = _emb_mm(x, W_emb, b_emb)
    agg_x = _seg_sum(x_in, src, dst, zeros)
    zx, h1 = _gin0_mm(x_in, agg_x, W_g0, b_g0)
    agg_1 = _seg_sum(h1, src, dst, zeros)
    h2 = _gin_mm(zx, h1, agg_1, W_g1[:_D], W_g1[_D:], b_g1)
    caggc, zxc, h2c = _center_stage(h2, zx, src, dst, center_node_index,
                                    zeros, zeros_i, trash_i)[1:]
    return _final_mm(zxc, h2c, caggc, W_g2[:_D], W_g2[_D:], b_g2,
                     W_o0, b_o0, W_o1, b_o1)


# speedup vs baseline: 1.9285x; 1.9285x over previous
import dataclasses
import functools

import jax
import jax.numpy as jnp
from jax import lax
from jax.experimental import pallas as pl
from jax.experimental.pallas import tpu as pltpu
from jax.experimental.pallas import tpu_sc as plsc

_N = 10000
_E = 320000
_D = 128
_NCTR = 512

_NCORES = 2
_NSUB = 16
_NW = _NCORES * _NSUB
_EPT = _E // _NW
_K = 128
_NFULL = _EPT // _K
_KTAIL = _EPT - _NFULL * _K
_RPT = 624
_RREM = _N - _NSUB * _RPT


def _sc_mesh():
    return plsc.VectorSubcoreMesh(
        core_axis_name="c", subcore_axis_name="s", num_cores=_NCORES,
        num_subcores=_NSUB)


def _sc_params(layout_passes=True):
    cp = pltpu.CompilerParams(disable_bounds_checks=True)
    if not layout_passes and (
            "needs_layout_passes" in pltpu.CompilerParams.__dataclass_fields__):
        cp = dataclasses.replace(cp, needs_layout_passes=False)
    return cp


def _seg_sum(h, src, dst, zeros):
    nrow = 2
    nidx = 4

    @functools.partial(
        pl.kernel,
        out_type=jax.ShapeDtypeStruct((_NCORES, _N, _D), jnp.float32),
        mesh=_sc_mesh(),
        compiler_params=_sc_params(),
        scratch_types=[
            [pltpu.VMEM((_K,), jnp.int32) for _ in range(nidx)],
            [pltpu.VMEM((_K,), jnp.int32) for _ in range(nidx)],
            [pltpu.VMEM((_K, _D), jnp.float32) for _ in range(nrow)],
            [pltpu.SemaphoreType.DMA for _ in range(nidx)],
            [pltpu.SemaphoreType.DMA for _ in range(nrow)],
            pltpu.VMEM((_KTAIL,), jnp.int32),
            pltpu.VMEM((_KTAIL,), jnp.int32),
            pltpu.VMEM((_KTAIL, _D), jnp.float32),
            pltpu.VMEM_SHARED((_N, _D), jnp.float32),
        ],
    )
    def k(h_hbm, src_hbm, dst_hbm, z_hbm, out_hbm, src_v, dst_v, rows_v,
          isem, gsem, srct_v, dstt_v, rowst_v, acc):
        cid = lax.axis_index("c")
        sid = lax.axis_index("s")
        wid = cid * _NSUB + sid
        r0 = pl.multiple_of(sid * _RPT, 8)
        pltpu.sync_copy(z_hbm.at[pl.ds(r0, _RPT)], acc.at[pl.ds(r0, _RPT)])

        @pl.when(sid == 0)
        def _():
            pltpu.sync_copy(z_hbm.at[pl.ds(_NSUB * _RPT, _RREM)],
                            acc.at[pl.ds(_NSUB * _RPT, _RREM)])

        plsc.subcore_barrier()
        base = wid * _EPT

        def issue_idx(c, bi):
            off = pl.multiple_of(base + c * _K, 8)
            pltpu.async_copy(src_hbm.at[pl.ds(off, _K)], src_v[bi], isem[bi])
            pltpu.async_copy(dst_hbm.at[pl.ds(off, _K)], dst_v[bi], isem[bi])

        def issue_gather(bi, br):
            pltpu.make_async_copy(src_hbm.at[pl.ds(0, _K)], src_v[bi],
                                  isem[bi]).wait()
            pltpu.make_async_copy(dst_hbm.at[pl.ds(0, _K)], dst_v[bi],
                                  isem[bi]).wait()
            pltpu.async_copy(h_hbm.at[src_v[bi]], rows_v[br], gsem[br])

        def drain(bi, br):
            pltpu.make_async_copy(h_hbm.at[src_v[bi]], rows_v[br],
                                  gsem[br]).wait()
            pltpu.sync_copy(rows_v[br], acc.at[dst_v[bi]], add=True)

        for c in range(nidx):
            issue_idx(c, c)
        issue_gather(0, 0)

        @pl.loop(0, 17)
        def _(j):
            c0 = 4 * j
            for b in range(4):
                issue_gather((b + 1) % nidx, (b + 1) % nrow)
                drain(b, b % nrow)
                issue_idx(c0 + b + 4, b)

        for c in range(68, _NFULL):
            if c + 1 < _NFULL:
                issue_gather((c + 1) % nidx, (c + 1) % nrow)
            drain(c % nidx, c % nrow)
            if c + 4 < _NFULL:
                issue_idx(c + 4, c % nidx)

        offt = pl.multiple_of(base + _NFULL * _K, 8)
        pltpu.sync_copy(src_hbm.at[pl.ds(offt, _KTAIL)], srct_v)
        pltpu.sync_copy(dst_hbm.at[pl.ds(offt, _KTAIL)], dstt_v)
        pltpu.sync_copy(h_hbm.at[srct_v], rowst_v)
        pltpu.sync_copy(rowst_v, acc.at[dstt_v], add=True)

        plsc.subcore_barrier()
        pltpu.sync_copy(acc.at[pl.ds(r0, _RPT)], out_hbm.at[cid, pl.ds(r0, _RPT)])

        @pl.when(sid == 0)
        def _():
            pltpu.sync_copy(acc.at[pl.ds(_NSUB * _RPT, _RREM)],
                            out_hbm.at[cid, pl.ds(_NSUB * _RPT, _RREM)])

    return k(h, src, dst, zeros)


_SCN = 1024
_SCNF = _EPT // _SCN
_SCNT = _EPT - _SCNF * _SCN
_MCAP = 10112
_CROWS = 520
_TRASH = _NCTR * 1024 + _NCTR


def _center_stage(h2, zx, src, dst, centers, zeros, zeros_i, trash_i):
    bpw = _NCTR // _NW
    bps = _NCTR // _NSUB

    @functools.partial(
        pl.kernel,
        compiler_params=_sc_params(layout_passes=False),
        out_type=[
            jax.ShapeDtypeStruct((_NCORES * _NCTR, _D), jnp.float32),
            jax.ShapeDtypeStruct((_NCORES, _NCTR, _D), jnp.float32),
            jax.ShapeDtypeStruct((_NCTR, _D), jnp.float32),
            jax.ShapeDtypeStruct((_NCTR, _D), jnp.float32),
        ],
        mesh=_sc_mesh(),
        scratch_types=[
            pltpu.VMEM((_N,), jnp.int32),
            pltpu.VMEM((_NCTR,), jnp.int32),
            [pltpu.VMEM((_SCN,), jnp.int32) for _ in range(2)],
            [pltpu.VMEM((_SCN,), jnp.int32) for _ in range(2)],
            [pltpu.SemaphoreType.DMA for _ in range(2)],
            pltpu.VMEM((_MCAP,), jnp.int32),
            pltpu.VMEM((_K,), jnp.int32),
            pltpu.VMEM((_K,), jnp.int32),
            pltpu.VMEM((_K, _D), jnp.float32),
            pltpu.VMEM((bps,), jnp.int32),
            pltpu.VMEM((bps, _D), jnp.float32),
            pltpu.VMEM((bps, _D), jnp.float32),
            pltpu.VMEM_SHARED((_CROWS, _D), jnp.float32),
        ],
    )
    def k(h2_hbm, zx_hbm, src_hbm, dst_hbm, ctr_hbm, z_hbm, zi_hbm, tr_hbm,
          raw_hbm, rem_hbm, zxc_hbm, h2c_hbm,
          marker, ctr_v, ssrc_v, sdst_v, ssem, mpack_v, gsrc_v,
          gdst_v, rows_v, pmap_v, prow_v, crow_v, cagg):
        cid = lax.axis_index("c")
        sid = lax.axis_index("s")
        wid = cid * _NSUB + sid

        pltpu.sync_copy(zi_hbm, marker)
        pltpu.sync_copy(tr_hbm, mpack_v)
        pltpu.sync_copy(ctr_hbm, ctr_v)
        for j in range(_NCTR // 16):
            cvec = ctr_v[pl.ds(16 * j, 16)]
            vals = lax.iota(jnp.int32, 16) + (16 * j + 1)
            plsc.store_scatter(marker, [cvec], vals)
        r0 = pl.multiple_of(sid * bps, 8)
        pltpu.sync_copy(z_hbm.at[pl.ds(r0, bps)], cagg.at[pl.ds(r0, bps)])

        @pl.when(sid == 0)
        def _():
            pltpu.sync_copy(z_hbm.at[pl.ds(_NCTR, _CROWS - _NCTR)],
                            cagg.at[pl.ds(_NCTR, _CROWS - _NCTR)])

        plsc.subcore_barrier()

        base = wid * _EPT

        def make_scan_sub(b):
            def scan_sub(u, cnt):
                dvec = sdst_v[b][pl.ds(16 * u, 16)]
                svec = ssrc_v[b][pl.ds(16 * u, 16)]
                m = plsc.load_gather(marker, [dvec])
                mask = m > 0
                packed = svec * 1024 + (m - 1)
                plsc.store_compressed(mpack_v.at[pl.ds(cnt, 16)], packed,
                                      mask=mask)
                return cnt + jnp.sum(mask.astype(jnp.int32))
            return scan_sub

        def issue_scan(ci, b):
            off = pl.multiple_of(base + ci * _SCN, 8)
            pltpu.async_copy(src_hbm.at[pl.ds(off, _SCN)], ssrc_v[b], ssem[b])
            pltpu.async_copy(dst_hbm.at[pl.ds(off, _SCN)], sdst_v[b], ssem[b])

        def wait_scan(b):
            pltpu.make_async_copy(src_hbm.at[pl.ds(0, _SCN)], ssrc_v[b],
                                  ssem[b]).wait()
            pltpu.make_async_copy(dst_hbm.at[pl.ds(0, _SCN)], sdst_v[b],
                                  ssem[b]).wait()

        def issue_scan_tail(b):
            offt = pl.multiple_of(base + _SCNF * _SCN, 8)
            pltpu.async_copy(src_hbm.at[pl.ds(offt, _SCNT)],
                             ssrc_v[b].at[pl.ds(0, _SCNT)], ssem[b])
            pltpu.async_copy(dst_hbm.at[pl.ds(offt, _SCNT)],
                             sdst_v[b].at[pl.ds(0, _SCNT)], ssem[b])

        def wait_scan_tail(b):
            pltpu.make_async_copy(src_hbm.at[pl.ds(0, _SCNT)],
                                  ssrc_v[b].at[pl.ds(0, _SCNT)],
                                  ssem[b]).wait()
            pltpu.make_async_copy(dst_hbm.at[pl.ds(0, _SCNT)],
                                  sdst_v[b].at[pl.ds(0, _SCNT)],
                                  ssem[b]).wait()

        issue_scan(0, 0)
        issue_scan(1, 1)
        cnt = jnp.int32(0)
        for ci in range(_SCNF):
            b = ci % 2
            wait_scan(b)
            cnt = lax.fori_loop(0, _SCN // 16, make_scan_sub(b), cnt)
            if ci + 2 < _SCNF:
                issue_scan(ci + 2, b)
            elif ci + 2 == _SCNF:
                issue_scan_tail(b)
        bt = _SCNF % 2
        wait_scan_tail(bt)
        cnt = lax.fori_loop(0, _SCNT // 16, make_scan_sub(bt), cnt)

        nch = (cnt + _K - 1) // _K

        @pl.loop(0, nch)
        def _(q):
            qo = q * _K
            for v in range(_K // 16):
                pk = mpack_v[pl.ds(qo + 16 * v, 16)]
                gsrc_v[pl.ds(16 * v, 16)] = pk >> 10
                gdst_v[pl.ds(16 * v, 16)] = pk & 1023
            pltpu.sync_copy(h2_hbm.at[gsrc_v], rows_v)
            pltpu.sync_copy(rows_v, cagg.at[gdst_v], add=True)

        plsc.subcore_barrier()

        pltpu.sync_copy(cagg.at[pl.ds(r0, bps)],
                        raw_hbm.at[pl.ds(cid * _NCTR + r0, bps)])
        plsc.subcore_barrier()

        pbase = pl.multiple_of(sid * bps, 8)
        for t in range(bps // 16):
            pvec = plsc.load_gather(
                marker, [ctr_v[pl.ds(pbase + 16 * t, 16)]]) - 1
            pmap_v[pl.ds(16 * t, 16)] = pvec + cid * _NCTR
        pltpu.sync_copy(raw_hbm.at[pmap_v], prow_v)
        pltpu.sync_copy(prow_v, rem_hbm.at[cid, pl.ds(pbase, bps)])

        @pl.when(cid == 0)
        def _():
            cbase = pl.multiple_of(sid * bps, 8)
            pltpu.sync_copy(zx_hbm.at[ctr_v.at[pl.ds(cbase, bps)]], crow_v)
            pltpu.sync_copy(crow_v, zxc_hbm.at[pl.ds(cbase, bps)])
            pltpu.sync_copy(h2_hbm.at[ctr_v.at[pl.ds(cbase, bps)]], crow_v)
            pltpu.sync_copy(crow_v, h2c_hbm.at[pl.ds(cbase, bps)])

    return k(h2, zx, src, dst, centers, zeros, zeros_i, trash_i)


_BR = 1000


def _emb_mm(x, W, b):
    def body(x_ref, w_ref, b_ref, o_ref):
        o_ref[...] = (
            jnp.dot(x_ref[...], w_ref[...], preferred_element_type=jnp.float32)
            + b_ref[...]
        )

    return pl.pallas_call(
        body,
        grid=(_N // _BR,),
        in_specs=[
            pl.BlockSpec((_BR, _D), lambda i: (i, 0)),
            pl.BlockSpec((_D, _D), lambda i: (0, 0)),
            pl.BlockSpec((1, _D), lambda i: (0, 0)),
        ],
        out_specs=pl.BlockSpec((_BR, _D), lambda i: (i, 0)),
        out_shape=jax.ShapeDtypeStruct((_N, _D), jnp.float32),
    )(x, W, b.reshape(1, _D))


def _gin0_mm(x_in, agg, W, b):
    def body(x_ref, a0_ref, a1_ref, w_ref, b_ref, z_ref, h_ref):
        z = x_ref[...] + a0_ref[0] + a1_ref[0]
        z_ref[...] = z
        h_ref[...] = jnp.maximum(
            jnp.dot(z, w_ref[...], preferred_element_type=jnp.float32)
            + b_ref[...],
            0.0,
        )

    return pl.pallas_call(
        body,
        grid=(_N // _BR,),
        in_specs=[
            pl.BlockSpec((_BR, _D), lambda i: (i, 0)),
            pl.BlockSpec((1, _BR, _D), lambda i: (0, i, 0)),
            pl.BlockSpec((1, _BR, _D), lambda i: (1, i, 0)),
            pl.BlockSpec((_D, _D), lambda i: (0, 0)),
            pl.BlockSpec((1, _D), lambda i: (0, 0)),
        ],
        out_specs=[
            pl.BlockSpec((_BR, _D), lambda i: (i, 0)),
            pl.BlockSpec((_BR, _D), lambda i: (i, 0)),
        ],
        out_shape=[
            jax.ShapeDtypeStruct((_N, _D), jnp.float32),
            jax.ShapeDtypeStruct((_N, _D), jnp.float32),
        ],
    )(x_in, agg, agg, W, b.reshape(1, _D))


def _gin_mm(zx, h, agg, Wa, Wb, b):
    def body(zx_ref, h_ref, a0_ref, a1_ref, wa_ref, wb_ref, b_ref, o_ref):
        zh = h_ref[...] + a0_ref[0] + a1_ref[0]
        acc = jnp.dot(zx_ref[...], wa_ref[...], preferred_element_type=jnp.float32)
        acc += jnp.dot(zh, wb_ref[...], preferred_element_type=jnp.float32)
        o_ref[...] = jnp.maximum(acc + b_ref[...], 0.0)

    return pl.pallas_call(
        body,
        grid=(_N // _BR,),
        in_specs=[
            pl.BlockSpec((_BR, _D), lambda i: (i, 0)),
            pl.BlockSpec((_BR, _D), lambda i: (i, 0)),
            pl.BlockSpec((1, _BR, _D), lambda i: (0, i, 0)),
            pl.BlockSpec((1, _BR, _D), lambda i: (1, i, 0)),
            pl.BlockSpec((_D, _D), lambda i: (0, 0)),
            pl.BlockSpec((_D, _D), lambda i: (0, 0)),
            pl.BlockSpec((1, _D), lambda i: (0, 0)),
        ],
        out_specs=pl.BlockSpec((_BR, _D), lambda i: (i, 0)),
        out_shape=jax.ShapeDtypeStruct((_N, _D), jnp.float32),
    )(zx, h, agg, agg, Wa, Wb, b.reshape(1, _D))


def _final_mm(zxc, h2c, caggc, Wa, Wb, bg, W0, b0, W1, b1):
    def body(zx_ref, h_ref, a0_ref, a1_ref, wa_ref, wb_ref, bg_ref,
             w0_ref, b0_ref, w1_ref, b1_ref, o_ref):
        zh = h_ref[...] + a0_ref[0] + a1_ref[0]
        acc = jnp.dot(zx_ref[...], wa_ref[...], preferred_element_type=jnp.float32)
        acc += jnp.dot(zh, wb_ref[...], preferred_element_type=jnp.float32)
        h3 = jnp.maximum(acc + bg_ref[...], 0.0)
        t = (
            jnp.dot(h3, w0_ref[...], preferred_element_type=jnp.float32)
            + b0_ref[...]
        )
        o_ref[...] = (
            jnp.dot(t, w1_ref[...], preferred_element_type=jnp.float32)
            + b1_ref[...]
        )

    return pl.pallas_call(
        body,
        grid=(1,),
        in_specs=[
            pl.BlockSpec((_NCTR, _D), lambda i: (0, 0)),
            pl.BlockSpec((_NCTR, _D), lambda i: (0, 0)),
            pl.BlockSpec((1, _NCTR, _D), lambda i: (0, 0, 0)),
            pl.BlockSpec((1, _NCTR, _D), lambda i: (1, 0, 0)),
            pl.BlockSpec((_D, _D), lambda i: (0, 0)),
            pl.BlockSpec((_D, _D), lambda i: (0, 0)),
            pl.BlockSpec((1, _D), lambda i: (0, 0)),
            pl.BlockSpec((_D, _D), lambda i: (0, 0)),
            pl.BlockSpec((1, _D), lambda i: (0, 0)),
            pl.BlockSpec((_D, 1), lambda i: (0, 0)),
            pl.BlockSpec((1, 1), lambda i: (0, 0)),
        ],
        out_specs=pl.BlockSpec((_NCTR, 1), lambda i: (0, 0)),
        out_shape=jax.ShapeDtypeStruct((_NCTR, 1), jnp.float32),
    )(zxc, h2c, caggc, caggc, Wa, Wb, bg.reshape(1, _D),
      W0, b0.reshape(1, _D), W1, b1.reshape(1, 1))


def kernel(x, edge_index, center_node_index, W_emb, b_emb, W_g0, b_g0,
           W_g1, b_g1, W_g2, b_g2, W_o0, b_o0, W_o1, b_o1):
    src = edge_index[0]
    dst = edge_index[1]
    zeros = jnp.zeros((_N, _D), jnp.float32)
    zeros_i = jnp.zeros((_N,), jnp.int32)
    trash_i = jnp.full((_MCAP,), _TRASH, jnp.int32)

    x_in = _emb_mm(x, W_emb, b_emb)
    agg_x = _seg_sum(x_in, src, dst, zeros)
    zx, h1 = _gin0_mm(x_in, agg_x, W_g0, b_g0)
    agg_1 = _seg_sum(h1, src, dst, zeros)
    h2 = _gin_mm(zx, h1, agg_1, W_g1[:_D], W_g1[_D:], b_g1)
    caggc, zxc, h2c = _center_stage(h2, zx, src, dst, center_node_index,
                                    zeros, zeros_i, trash_i)[1:]
    return _final_mm(zxc, h2c, caggc, W_g2[:_D], W_g2[_D:], b_g2,
                     W_o0, b_o0, W_o1, b_o1)
